# Initial kernel scaffold; baseline (speedup 1.0000x reference)
import jax
import jax.numpy as jnp
from jax.experimental import pallas as pl


def kernel(indices, weight):
    raise NotImplementedError("write your pallas kernel here")


# speedup vs baseline: 1.5692x; 1.5692x over previous
import jax
import jax.numpy as jnp
from jax import lax
from jax.experimental import pallas as pl
from jax.experimental.pallas import tpu as pltpu, tpu_sc as plsc

NCB = 8
V = 100000
D = 64
N = 1024 * 50
NW = 32
PER_W = N * NCB // NW
CHUNK = 128
NCHUNK = PER_W // CHUNK
GROUP = 20
NGROUP = NCHUNK // GROUP
POS_PER_CHUNK = CHUNK // NCB
POS_PER_GROUP = GROUP * POS_PER_CHUNK


def _body(idx_hbm, w_hbm, out_hbm, idx_buf, rows_buf, out_buf, sem):
    nc = 2
    wid = lax.axis_index("s") * nc + lax.axis_index("c")

    pltpu.sync_copy(idx_hbm.at[wid], idx_buf)
    offs = (jnp.arange(16, dtype=jnp.int32) % NCB) * V

    def add_offs(k, _):
        for j in range(CHUNK // 16):
            sl = pl.ds(j * 16, 16)
            idx_buf[k, sl] = idx_buf[k, sl] + offs
        return 0

    lax.fori_loop(0, NCHUNK, add_offs, 0)

    out_base = wid * (PER_W // NCB)

    for g in range(NGROUP):
        def chunk_body(k, _):
            pltpu.async_copy(
                w_hbm.at[idx_buf.at[g * GROUP + k]], rows_buf, sem
            ).wait()

            def pos_body(p, _):
                r0 = NCB * p
                orow = k * POS_PER_CHUNK + p
                for j in range(D // 16):
                    sl = pl.ds(j * 16, 16)
                    acc = rows_buf[r0, sl]
                    for c in range(1, NCB):
                        acc = acc + rows_buf[r0 + c, sl]
                    out_buf[orow, sl] = acc
                return 0

            lax.fori_loop(0, POS_PER_CHUNK, pos_body, 0)
            return 0

        lax.fori_loop(0, GROUP, chunk_body, 0)
        pltpu.sync_copy(
            out_buf,
            out_hbm.at[pl.ds(out_base + g * POS_PER_GROUP, POS_PER_GROUP)],
        )


def _run(idx3, wflat):
    f = pl.kernel(
        _body,
        out_type=jax.ShapeDtypeStruct((N, D), jnp.float32),
        mesh=plsc.VectorSubcoreMesh(core_axis_name="c", subcore_axis_name="s"),
        scratch_types=[
            pltpu.VMEM((NCHUNK, CHUNK), jnp.int32),
            pltpu.VMEM((CHUNK, D), jnp.float32),
            pltpu.VMEM((POS_PER_GROUP, D), jnp.float32),
            pltpu.SemaphoreType.DMA,
        ],
        compiler_params=pltpu.CompilerParams(use_tc_tiling_on_sc=False),
    )
    return f(idx3, wflat)


@jax.jit
def kernel(indices, weight):
    B, S, C = indices.shape
    idx3 = indices.astype(jnp.int32).reshape(NW, NCHUNK, CHUNK)
    wflat = weight.reshape(C * V, D)
    out = _run(idx3, wflat)
    return out.reshape(B, S, D)


# speedup vs baseline: 1.7784x; 1.1333x over previous
import jax
import jax.numpy as jnp
from jax import lax
from jax.experimental import pallas as pl
from jax.experimental.pallas import tpu as pltpu, tpu_sc as plsc

NCB = 8
V = 100000
D = 64
N = 1024 * 50
NW = 32
PER_W = N * NCB // NW
CHUNK = 128
NCHUNK = PER_W // CHUNK
GROUP = 20
NGROUP = NCHUNK // GROUP
POS_PER_CHUNK = CHUNK // NCB
POS_PER_GROUP = GROUP * POS_PER_CHUNK
NBUF = 2


def _body(idx_hbm, w_hbm, out_hbm, idx_buf, rows_buf, out_buf, sem0, sem1):
    nc = 2
    wid = lax.axis_index("s") * nc + lax.axis_index("c")

    pltpu.sync_copy(idx_hbm.at[wid], idx_buf)
    offs = (jnp.arange(16, dtype=jnp.int32) % NCB) * V

    def add_offs(k, _):
        for j in range(CHUNK // 16):
            sl = pl.ds(j * 16, 16)
            idx_buf[k, sl] = idx_buf[k, sl] + offs
        return 0

    lax.fori_loop(0, NCHUNK, add_offs, 0)

    sems = (sem0, sem1)

    for b in range(NBUF):
        pltpu.async_copy(w_hbm.at[idx_buf.at[b]], rows_buf.at[b], sems[b])

    out_base = wid * (PER_W // NCB)
    npair = GROUP // NBUF

    for g in range(NGROUP):
        def pair_body(i, _):
            for b in range(NBUF):
                k = g * GROUP + i * NBUF + b
                pltpu.make_async_copy(
                    w_hbm.at[idx_buf.at[k]], rows_buf.at[b], sems[b]
                ).wait()

                def pos_body(p, _):
                    r0 = NCB * p
                    orow = (i * NBUF + b) * POS_PER_CHUNK + p
                    for j in range(D // 16):
                        sl = pl.ds(j * 16, 16)
                        acc = rows_buf[b, r0, sl]
                        for c in range(1, NCB):
                            acc = acc + rows_buf[b, r0 + c, sl]
                        out_buf[orow, sl] = acc
                    return 0

                lax.fori_loop(0, POS_PER_CHUNK, pos_body, 0)

                @pl.when(k + NBUF < NCHUNK)
                def _():
                    pltpu.async_copy(
                        w_hbm.at[idx_buf.at[k + NBUF]], rows_buf.at[b], sems[b]
                    )

            return 0

        lax.fori_loop(0, npair, pair_body, 0)
        pltpu.sync_copy(
            out_buf,
            out_hbm.at[pl.ds(out_base + g * POS_PER_GROUP, POS_PER_GROUP)],
        )


def _run(idx3, wflat):
    f = pl.kernel(
        _body,
        out_type=jax.ShapeDtypeStruct((N, D), jnp.float32),
        mesh=plsc.VectorSubcoreMesh(core_axis_name="c", subcore_axis_name="s"),
        scratch_types=[
            pltpu.VMEM((NCHUNK, CHUNK), jnp.int32),
            pltpu.VMEM((NBUF, CHUNK, D), jnp.float32),
            pltpu.VMEM((POS_PER_GROUP, D), jnp.float32),
            pltpu.SemaphoreType.DMA,
            pltpu.SemaphoreType.DMA,
        ],
        compiler_params=pltpu.CompilerParams(use_tc_tiling_on_sc=False),
    )
    return f(idx3, wflat)


@jax.jit
def kernel(indices, weight):
    B, S, C = indices.shape
    idx3 = indices.astype(jnp.int32).reshape(NW, NCHUNK, CHUNK)
    wflat = weight.reshape(C * V, D)
    out = _run(idx3, wflat)
    return out.reshape(B, S, D)


# speedup vs baseline: 1.9881x; 1.1179x over previous
import jax
import jax.numpy as jnp
from jax import lax
from jax.experimental import pallas as pl
from jax.experimental.pallas import tpu as pltpu, tpu_sc as plsc

NCB = 8
V = 100000
D = 64
N = 1024 * 50
NW = 32
POS_W = N // NW
CHUNK = 64
NCHUNK = POS_W // CHUNK
NPAIR = (NCHUNK - 1) // 2


def _body(idx_hbm, w_hbm, out_hbm, idx_buf, rows_buf, ob, gsem0, gsem1,
          osem0, osem1):
    nc = 2
    wid = lax.axis_index("s") * nc + lax.axis_index("c")
    out_base = wid * POS_W

    pltpu.sync_copy(idx_hbm.at[wid], idx_buf)

    gsems = (gsem0, gsem1)
    osems = (osem0, osem1)

    def issue(k, b):
        for c in range(NCB):
            pltpu.async_copy(
                w_hbm.at[c].at[idx_buf.at[c, k]], rows_buf.at[b, c], gsems[b]
            )

    def drain_gathers(b):
        for c in range(NCB):
            pltpu.make_async_copy(
                w_hbm.at[c].at[idx_buf.at[c, 0]], rows_buf.at[b, c], gsems[b]
            ).wait()

    def accumulate(b):
        def pos_body(p, _):
            for j in range(D // 16):
                sl = pl.ds(j * 16, 16)
                acc = rows_buf[b, 0, p, sl]
                for c in range(1, NCB):
                    acc = acc + rows_buf[b, c, p, sl]
                ob[b, p, sl] = acc
            return 0

        lax.fori_loop(0, CHUNK, pos_body, 0)

    def flush(k, b):
        pltpu.async_copy(
            ob.at[b], out_hbm.at[pl.ds(out_base + k * CHUNK, CHUNK)], osems[b]
        )

    def drain_flush(b):
        pltpu.make_async_copy(
            out_hbm.at[pl.ds(0, CHUNK)], ob.at[b], osems[b]
        ).wait()

    issue(0, 0)

    def pair_body(i, _):
        k = 2 * i
        issue(k + 1, 1)
        drain_gathers(0)

        @pl.when(i > 0)
        def _():
            drain_flush(0)

        accumulate(0)
        flush(k, 0)
        issue(k + 2, 0)
        drain_gathers(1)

        @pl.when(i > 0)
        def _():
            drain_flush(1)

        accumulate(1)
        flush(k + 1, 1)
        return 0

    lax.fori_loop(0, NPAIR, pair_body, 0)

    drain_gathers(0)
    drain_flush(0)
    accumulate(0)
    flush(NCHUNK - 1, 0)
    drain_flush(1)
    drain_flush(0)


def _run(idx4, weight):
    f = pl.kernel(
        _body,
        out_type=jax.ShapeDtypeStruct((N, D), jnp.float32),
        mesh=plsc.VectorSubcoreMesh(core_axis_name="c", subcore_axis_name="s"),
        scratch_types=[
            pltpu.VMEM((NCB, NCHUNK, CHUNK), jnp.int32),
            pltpu.VMEM((2, NCB, CHUNK, D), jnp.float32),
            pltpu.VMEM((2, CHUNK, D), jnp.float32),
            pltpu.SemaphoreType.DMA,
            pltpu.SemaphoreType.DMA,
            pltpu.SemaphoreType.DMA,
            pltpu.SemaphoreType.DMA,
        ],
        compiler_params=pltpu.CompilerParams(use_tc_tiling_on_sc=False),
    )
    return f(idx4, weight)


@jax.jit
def kernel(indices, weight):
    B, S, C = indices.shape
    idx4 = (
        indices.astype(jnp.int32)
        .reshape(NW, POS_W, C)
        .transpose(0, 2, 1)
        .reshape(NW, C, NCHUNK, CHUNK)
    )
    out = _run(idx4, weight)
    return out.reshape(B, S, D)
